# baseline (speedup 1.0000x reference)
import numpy as np

import jax
import jax.numpy as jnp
from jax.experimental import pallas as pl
from jax.experimental.pallas import tpu as pltpu

_INTERPRET = False

_BT = 1024
_BN = 1024
_MASKED = -150.0


def _proj_kernel(w_ref, x_ref, out_ref):
    out_ref[...] = jnp.tanh(
        jax.lax.dot_general(
            w_ref[...], x_ref[...], (((1,), (1,)), ((), ())),
            preferred_element_type=jnp.float32,
        )
    )


def _step_kernel(im_ref, jm_ref, act_ref, pi_ref, pj2_ref, sqe_ref, sqr_ref,
                 rsq_ref, rr_ref, aug_ref, augt_ref, ai_ref, thr_ref, out_ref,
                 z_ref, zt_ref):
    t = pl.program_id(0)
    i = im_ref[t]
    j = jm_ref[t]
    nj = pl.num_programs(0)
    b = out_ref.shape[1]
    bt = _BT
    nblk = aug_ref.shape[0] // bt

    @pl.when(t == 0)
    def _():
        z_ref[...] = jnp.zeros_like(z_ref)
        zt_ref[...] = jnp.zeros_like(zt_ref)

    def weights(masked_diag):
        g2 = jax.lax.dot_general(
            pi_ref[...], pj2_ref[...], (((1,), (0,)), ((), ())),
            preferred_element_type=jnp.float32,
        )
        d2 = jnp.maximum((sqe_ref[...] + sqr_ref[...]) - g2, 1e-9)
        arg = jnp.where(d2 <= rsq_ref[...],
                        d2 * jax.lax.rsqrt(d2) * rr_ref[...], _MASKED)
        if masked_diag:
            neq = (jax.lax.broadcasted_iota(jnp.int32, (bt, bt), 0)
                   != jax.lax.broadcasted_iota(jnp.int32, (bt, bt), 1))
            arg = jnp.where(neq, arg, _MASKED)
        return jnp.exp2(arg).astype(jnp.bfloat16)

    @pl.when((j > i) & (act_ref[t] != 0))
    def _():
        w_bf = weights(False)
        z_ref[pl.ds(i * bt, bt), :] += jax.lax.dot_general(
            w_bf, aug_ref[pl.ds(j * bt, bt), :], (((1,), (0,)), ((), ())),
            preferred_element_type=jnp.float32,
        )
        zt_ref[:, pl.ds(j * bt, bt)] += jax.lax.dot_general(
            augt_ref[:, pl.ds(i * bt, bt)], w_bf, (((1,), (0,)), ((), ())),
            preferred_element_type=jnp.float32,
        )

    @pl.when(j == i)
    def _():
        w_bf = weights(True)
        z_ref[pl.ds(i * bt, bt), :] += jax.lax.dot_general(
            w_bf, aug_ref[pl.ds(i * bt, bt), :], (((1,), (0,)), ((), ())),
            preferred_element_type=jnp.float32,
        )

    @pl.when(j == nblk - 1)
    def _():
        zi = (z_ref[pl.ds(i * bt, bt), :]
              + jnp.transpose(zt_ref[:, pl.ds(i * bt, bt)], (1, 0)))
        denom = zi[:, b:b + 1] + 1e-8
        out_ref[...] = jnp.tanh(zi[:, :b] / denom + ai_ref[...] - thr_ref[...])


def _out_kernel(a_ref, wo_ref, o_ref):
    o_ref[...] = jax.lax.dot_general(
        a_ref[...], wo_ref[...], (((0,), (0,)), ((), ())),
        preferred_element_type=jnp.float32,
    )


def kernel(x, positions, input_weights, features, output_weights,
           connection_radii, thresholds, n_iterations):
    n = positions.shape[0]
    b, in_sz = x.shape
    o_sz = output_weights.shape[1]
    nblk = n // _BT

    order = jnp.argsort(positions[:, 0])
    positions = positions[order]
    connection_radii = connection_radii[order]
    thresholds = thresholds[order]

    pos_t2 = (positions * 2.0).T
    sq = jnp.sum(positions * positions, axis=1)
    sqe_col = (sq + 1e-9)[:, None]
    sq_row = sq[None, :]
    rsq_col = (connection_radii * connection_radii)[:, None]
    log2e = 1.4426950408889634
    rr_col = (-log2e / connection_radii)[:, None]
    thr_col = thresholds[:, None]
    ones_col = jnp.ones((n, 1), jnp.float32)

    pairs = [(i, j) for i in range(nblk) for j in range(i, nblk)]
    im = jnp.asarray(np.array([p[0] for p in pairs], dtype=np.int32))
    jm = jnp.asarray(np.array([p[1] for p in pairs], dtype=np.int32))

    pb = positions.reshape(nblk, _BT, 3)
    bmin = pb.min(axis=1)
    bmax = pb.max(axis=1)
    gap = jnp.maximum(jnp.maximum(bmin[None, :, :] - bmax[:, None, :],
                                  bmin[:, None, :] - bmax[None, :, :]), 0.0)
    gap2 = jnp.sum(gap * gap, axis=-1)
    reach = (jnp.max(connection_radii) + 0.5) ** 2
    act = jnp.ones((len(pairs),), jnp.int32)

    a_t = pl.pallas_call(
        _proj_kernel,
        grid=(n // _BN,),
        in_specs=[
            pl.BlockSpec((_BN, in_sz), lambda i: (i, 0)),
            pl.BlockSpec((b, in_sz), lambda i: (0, 0)),
        ],
        out_specs=pl.BlockSpec((_BN, b), lambda i: (i, 0)),
        out_shape=jax.ShapeDtypeStruct((n, b), jnp.float32),
        interpret=_INTERPRET,
    )(input_weights, x)

    step = pl.pallas_call(
        _step_kernel,
        grid_spec=pltpu.PrefetchScalarGridSpec(
            num_scalar_prefetch=3,
            grid=(len(pairs),),
            in_specs=[
                pl.BlockSpec((_BT, 3), lambda t, im, jm, act: (im[t], 0)),
                pl.BlockSpec((3, _BT), lambda t, im, jm, act: (0, jm[t])),
                pl.BlockSpec((_BT, 1), lambda t, im, jm, act: (im[t], 0)),
                pl.BlockSpec((1, _BT), lambda t, im, jm, act: (0, jm[t])),
                pl.BlockSpec((_BT, 1), lambda t, im, jm, act: (im[t], 0)),
                pl.BlockSpec((_BT, 1), lambda t, im, jm, act: (im[t], 0)),
                pl.BlockSpec((n, b + 1), lambda t, im, jm, act: (0, 0)),
                pl.BlockSpec((b + 1, n), lambda t, im, jm, act: (0, 0)),
                pl.BlockSpec((_BT, b), lambda t, im, jm, act: (im[t], 0)),
                pl.BlockSpec((_BT, 1), lambda t, im, jm, act: (im[t], 0)),
            ],
            out_specs=pl.BlockSpec((_BT, b),
                                   lambda t, im, jm, act: (im[t], 0)),
            scratch_shapes=[
                pltpu.VMEM((n, b + 1), jnp.float32),
                pltpu.VMEM((b + 1, n), jnp.float32),
            ],
        ),
        out_shape=jax.ShapeDtypeStruct((n, b), jnp.float32),
        compiler_params=pltpu.CompilerParams(
            dimension_semantics=("arbitrary",),
        ),
        interpret=_INTERPRET,
    )

    def body(_, a):
        a_aug = jnp.concatenate([a, ones_col], axis=1).astype(jnp.bfloat16)
        return step(im, jm, act, positions, pos_t2, sqe_col, sq_row, rsq_col,
                    rr_col, a_aug, a_aug.T, a, thr_col)

    a_t = jax.lax.fori_loop(0, n_iterations, body, a_t[order])

    out = pl.pallas_call(
        _out_kernel,
        in_specs=[
            pl.BlockSpec((n, b), lambda: (0, 0)),
            pl.BlockSpec((n, o_sz), lambda: (0, 0)),
        ],
        out_specs=pl.BlockSpec((b, o_sz), lambda: (0, 0)),
        out_shape=jax.ShapeDtypeStruct((b, o_sz), jnp.float32),
        interpret=_INTERPRET,
    )(a_t, output_weights[order])

    return out


# speedup vs baseline: 1.2954x; 1.2954x over previous
import numpy as np

import jax
import jax.numpy as jnp
from jax.experimental import pallas as pl
from jax.experimental.pallas import tpu as pltpu

_INTERPRET = False

_BT = 1024
_BN = 1024
_MASKED = -150.0


def _proj_kernel(w_ref, x_ref, out_ref):
    out_ref[...] = jnp.tanh(
        jax.lax.dot_general(
            w_ref[...], x_ref[...], (((1,), (1,)), ((), ())),
            preferred_element_type=jnp.float32,
        )
    )


def _step_kernel(im_ref, jm_ref, act_ref, pk_ref, pj2_ref, sqr_ref,
                 aug_ref, augt_ref, a_ref, out_ref, z_ref, zt_ref):
    t = pl.program_id(0)
    i = im_ref[t]
    j = jm_ref[t]
    b = out_ref.shape[1]
    bt = _BT
    nblk = out_ref.shape[0] // bt

    @pl.when(t == 0)
    def _():
        z_ref[...] = jnp.zeros_like(z_ref)
        zt_ref[...] = jnp.zeros_like(zt_ref)

    def weights(masked_diag):
        pki = pk_ref[pl.ds(i * bt, bt), :]
        g2 = jax.lax.dot_general(
            pki[:, 0:3], pj2_ref[:, pl.ds(j * bt, bt)],
            (((1,), (0,)), ((), ())),
            preferred_element_type=jnp.float32,
        )
        d2 = jnp.maximum(
            (pki[:, 3:4] + sqr_ref[:, pl.ds(j * bt, bt)]) - g2, 1e-9)
        arg = jnp.where(d2 <= pki[:, 4:5],
                        d2 * jax.lax.rsqrt(d2) * pki[:, 5:6], _MASKED)
        if masked_diag:
            neq = (jax.lax.broadcasted_iota(jnp.int32, (bt, bt), 0)
                   != jax.lax.broadcasted_iota(jnp.int32, (bt, bt), 1))
            arg = jnp.where(neq, arg, _MASKED)
        return jnp.exp2(arg).astype(jnp.bfloat16)

    @pl.when((j > i) & (act_ref[t] != 0))
    def _():
        w_bf = weights(False)
        z_ref[pl.ds(i * bt, bt), :] += jax.lax.dot_general(
            w_bf, aug_ref[pl.ds(j * bt, bt), :], (((1,), (0,)), ((), ())),
            preferred_element_type=jnp.float32,
        )
        zt_ref[:, pl.ds(j * bt, bt)] += jax.lax.dot_general(
            augt_ref[:, pl.ds(i * bt, bt)], w_bf, (((1,), (0,)), ((), ())),
            preferred_element_type=jnp.float32,
        )

    @pl.when(j == i)
    def _():
        w_bf = weights(True)
        z_ref[pl.ds(i * bt, bt), :] += jax.lax.dot_general(
            w_bf, aug_ref[pl.ds(i * bt, bt), :], (((1,), (0,)), ((), ())),
            preferred_element_type=jnp.float32,
        )

    @pl.when(j == nblk - 1)
    def _():
        zi = (z_ref[pl.ds(i * bt, bt), :]
              + jnp.transpose(zt_ref[:, pl.ds(i * bt, bt)], (1, 0)))
        denom = zi[:, b:b + 1] + 1e-8
        out_ref[pl.ds(i * bt, bt), :] = jnp.tanh(
            zi[:, :b] / denom + a_ref[pl.ds(i * bt, bt), :]
            - pk_ref[pl.ds(i * bt, bt), 6:7])


def _out_kernel(a_ref, wo_ref, o_ref):
    o_ref[...] = jax.lax.dot_general(
        a_ref[...], wo_ref[...], (((0,), (0,)), ((), ())),
        preferred_element_type=jnp.float32,
    )


def kernel(x, positions, input_weights, features, output_weights,
           connection_radii, thresholds, n_iterations):
    n = positions.shape[0]
    b, in_sz = x.shape
    o_sz = output_weights.shape[1]
    nblk = n // _BT

    order = jnp.argsort(positions[:, 0])
    positions = positions[order]
    connection_radii = connection_radii[order]
    thresholds = thresholds[order]

    pos_t2 = (positions * 2.0).T
    sq = jnp.sum(positions * positions, axis=1)
    sq_row = sq[None, :]
    log2e = 1.4426950408889634
    packed = jnp.stack(
        [positions[:, 0], positions[:, 1], positions[:, 2],
         sq + 1e-9,
         connection_radii * connection_radii,
         -log2e / connection_radii,
         thresholds,
         jnp.zeros((n,), jnp.float32)], axis=1)
    ones_col = jnp.ones((n, 1), jnp.float32)

    pairs = [(i, j) for i in range(nblk) for j in range(i, nblk)]
    im = jnp.asarray(np.array([p[0] for p in pairs], dtype=np.int32))
    jm = jnp.asarray(np.array([p[1] for p in pairs], dtype=np.int32))

    pb = positions.reshape(nblk, _BT, 3)
    bmin = pb.min(axis=1)
    bmax = pb.max(axis=1)
    gap = jnp.maximum(jnp.maximum(bmin[None, :, :] - bmax[:, None, :],
                                  bmin[:, None, :] - bmax[None, :, :]), 0.0)
    gap2 = jnp.sum(gap * gap, axis=-1)
    reach = (jnp.max(connection_radii) + 0.5) ** 2
    act = (gap2[im, jm] <= reach).astype(jnp.int32)

    a_t = pl.pallas_call(
        _proj_kernel,
        grid=(n // _BN,),
        in_specs=[
            pl.BlockSpec((_BN, in_sz), lambda i: (i, 0)),
            pl.BlockSpec((b, in_sz), lambda i: (0, 0)),
        ],
        out_specs=pl.BlockSpec((_BN, b), lambda i: (i, 0)),
        out_shape=jax.ShapeDtypeStruct((n, b), jnp.float32),
        interpret=_INTERPRET,
    )(input_weights, x)

    step = pl.pallas_call(
        _step_kernel,
        grid_spec=pltpu.PrefetchScalarGridSpec(
            num_scalar_prefetch=3,
            grid=(len(pairs),),
            in_specs=[
                pl.BlockSpec((n, 8), lambda t, im, jm, act: (0, 0)),
                pl.BlockSpec((3, n), lambda t, im, jm, act: (0, 0)),
                pl.BlockSpec((1, n), lambda t, im, jm, act: (0, 0)),
                pl.BlockSpec((n, b + 1), lambda t, im, jm, act: (0, 0)),
                pl.BlockSpec((b + 1, n), lambda t, im, jm, act: (0, 0)),
                pl.BlockSpec((n, b), lambda t, im, jm, act: (0, 0)),
            ],
            out_specs=pl.BlockSpec((n, b),
                                   lambda t, im, jm, act: (0, 0)),
            scratch_shapes=[
                pltpu.VMEM((n, b + 1), jnp.float32),
                pltpu.VMEM((b + 1, n), jnp.float32),
            ],
        ),
        out_shape=jax.ShapeDtypeStruct((n, b), jnp.float32),
        compiler_params=pltpu.CompilerParams(
            dimension_semantics=("arbitrary",),
        ),
        interpret=_INTERPRET,
    )

    def body(_, a):
        a_aug = jnp.concatenate([a, ones_col], axis=1).astype(jnp.bfloat16)
        return step(im, jm, act, packed, pos_t2, sq_row,
                    a_aug, a_aug.T, a)

    a_t = jax.lax.fori_loop(0, n_iterations, body, a_t[order])

    out = pl.pallas_call(
        _out_kernel,
        in_specs=[
            pl.BlockSpec((n, b), lambda: (0, 0)),
            pl.BlockSpec((n, o_sz), lambda: (0, 0)),
        ],
        out_specs=pl.BlockSpec((b, o_sz), lambda: (0, 0)),
        out_shape=jax.ShapeDtypeStruct((b, o_sz), jnp.float32),
        interpret=_INTERPRET,
    )(a_t, output_weights[order])

    return out
